# baseline (speedup 1.0000x reference)
import functools

import jax
import jax.numpy as jnp
from jax import lax
from jax.experimental import pallas as pl
from jax.experimental.pallas import tpu as pltpu
from jax.experimental.pallas import tpu_sc as plsc

_T, _D, _H, _E, _GH, _K = 2048, 1024, 2048, 8, 256, 2
_BT = 256
_BR = 128
_LP = _T * _K + _E * _BR
_NB = _LP // _BR
_NEG = -1e30


def _gating_body(x_ref, gw1_ref, gb1_ref, gw2_ref, gb2_ref, idx_ref, w_ref):
    x = x_ref[...]
    h = jnp.maximum(
        jnp.dot(x, gw1_ref[...], preferred_element_type=jnp.float32)
        + gb1_ref[...], 0.0)
    logits = (jnp.dot(h, gw2_ref[...], preferred_element_type=jnp.float32)
              + gb2_ref[...])
    lane = lax.broadcasted_iota(jnp.int32, logits.shape, 1)
    m1 = jnp.max(logits, axis=-1, keepdims=True)
    i1 = jnp.min(jnp.where(logits == m1, lane, 127), axis=-1, keepdims=True)
    l2 = jnp.where(lane == i1, _NEG, logits)
    m2 = jnp.max(l2, axis=-1, keepdims=True)
    i2 = jnp.min(jnp.where(l2 == m2, lane, 127), axis=-1, keepdims=True)
    e21 = jnp.exp(m2 - m1)
    w1v = 1.0 / (1.0 + e21)
    idx_ref[...] = jnp.concatenate([i1, i2], axis=1)
    w_ref[...] = jnp.concatenate([w1v, 1.0 - w1v], axis=1)


def _gating(x, gw1, gb1, gw2, gb2):
    gw2p = jnp.pad(gw2, ((0, 0), (0, 128 - _E)))
    gb2p = jnp.pad(gb2, (0, 128 - _E), constant_values=_NEG).reshape(1, 128)
    gb1r = gb1.reshape(1, _GH)
    grid = (_T // _BT,)
    return pl.pallas_call(
        _gating_body,
        grid=grid,
        in_specs=[
            pl.BlockSpec((_BT, _D), lambda i: (i, 0)),
            pl.BlockSpec((_D, _GH), lambda i: (0, 0)),
            pl.BlockSpec((1, _GH), lambda i: (0, 0)),
            pl.BlockSpec((_GH, 128), lambda i: (0, 0)),
            pl.BlockSpec((1, 128), lambda i: (0, 0)),
        ],
        out_specs=[
            pl.BlockSpec((_BT, _K), lambda i: (i, 0)),
            pl.BlockSpec((_BT, _K), lambda i: (i, 0)),
        ],
        out_shape=[
            jax.ShapeDtypeStruct((_T, _K), jnp.int32),
            jax.ShapeDtypeStruct((_T, _K), jnp.float32),
        ],
    )(x, gw1, gb1r, gw2p, gb2p)


def _ffn_body(be_ref, xs_ref, w1_ref, b1_ref, w2_ref, b2_ref, ws_ref, y_ref):
    x = xs_ref[...]
    a = jnp.tanh(
        jnp.dot(x, w1_ref[0], preferred_element_type=jnp.float32)
        + b1_ref[0])
    y = (jnp.dot(a, w2_ref[0], preferred_element_type=jnp.float32)
         + b2_ref[0])
    y_ref[...] = y * ws_ref[...]


def _ffn(x_sorted, w_sorted, block_expert, exp_w1, exp_b1, exp_w2, exp_b2):
    grid_spec = pltpu.PrefetchScalarGridSpec(
        num_scalar_prefetch=1,
        grid=(_NB,),
        in_specs=[
            pl.BlockSpec((_BR, _D), lambda b, be: (b, 0)),
            pl.BlockSpec((1, _D, _H), lambda b, be: (be[b], 0, 0)),
            pl.BlockSpec((1, 1, _H), lambda b, be: (be[b], 0, 0)),
            pl.BlockSpec((1, _H, _D), lambda b, be: (be[b], 0, 0)),
            pl.BlockSpec((1, 1, _D), lambda b, be: (be[b], 0, 0)),
            pl.BlockSpec((_BR, 1), lambda b, be: (b, 0)),
        ],
        out_specs=pl.BlockSpec((_BR, _D), lambda b, be: (b, 0)),
    )
    return pl.pallas_call(
        _ffn_body,
        grid_spec=grid_spec,
        out_shape=jax.ShapeDtypeStruct((_LP, _D), jnp.float32),
    )(block_expert, x_sorted, exp_w1, exp_b1.reshape(_E, 1, _H),
      exp_w2, exp_b2.reshape(_E, 1, _D), w_sorted.reshape(_LP, 1))


_NW = 32
_DCH = 40
_GCH = 64


def _sc_wid():
    return lax.axis_index("s") * 2 + lax.axis_index("c")


def _dispatch_gather(x, tok_sorted):
    rows_per_w = _LP // _NW
    nch = rows_per_w // _DCH

    @functools.partial(
        pl.kernel,
        out_type=jax.ShapeDtypeStruct((_LP, _D), jnp.float32),
        mesh=plsc.VectorSubcoreMesh(core_axis_name="c", subcore_axis_name="s"),
        scratch_types=[
            pltpu.VMEM((_DCH,), jnp.int32),
            pltpu.VMEM((_DCH,), jnp.int32),
            pltpu.VMEM((_DCH, _D), jnp.float32),
            pltpu.VMEM((_DCH, _D), jnp.float32),
            pltpu.SemaphoreType.DMA,
            pltpu.SemaphoreType.DMA,
        ],
    )
    def k(x_hbm, idx_hbm, out_hbm, i0, i1, r0, r1, s0, s1):
        wid = _sc_wid()
        idx_v, rows_v, sems = [i0, i1], [r0, r1], [s0, s1]
        base0 = wid * rows_per_w
        pltpu.sync_copy(idx_hbm.at[pl.ds(base0, _DCH)], i0)
        cps = [pltpu.async_copy(x_hbm.at[i0], r0, s0), None]
        for c in range(1, nch):
            b = base0 + c * _DCH
            pltpu.sync_copy(idx_hbm.at[pl.ds(b, _DCH)], idx_v[c % 2])
            cps[c % 2] = pltpu.async_copy(
                x_hbm.at[idx_v[c % 2]], rows_v[c % 2], sems[c % 2])
            cps[(c - 1) % 2].wait()
            pltpu.sync_copy(rows_v[(c - 1) % 2],
                            out_hbm.at[pl.ds(b - _DCH, _DCH)])
        cps[(nch - 1) % 2].wait()
        pltpu.sync_copy(rows_v[(nch - 1) % 2],
                        out_hbm.at[pl.ds(base0 + (nch - 1) * _DCH, _DCH)])

    return k(x, tok_sorted)


def _combine_gather(y_sorted, pos_cat):
    rows_per_w = (_K * _T) // _NW

    @functools.partial(
        pl.kernel,
        out_type=jax.ShapeDtypeStruct((_K * _T, _D), jnp.float32),
        mesh=plsc.VectorSubcoreMesh(core_axis_name="c", subcore_axis_name="s"),
        scratch_types=[
            pltpu.VMEM((_GCH,), jnp.int32),
            pltpu.VMEM((_GCH, _D), jnp.float32),
            pltpu.SemaphoreType.DMA,
        ],
    )
    def k(y_hbm, idx_hbm, out_hbm, idx_v, rows_v, sem):
        wid = _sc_wid()
        for c in range(rows_per_w // _GCH):
            base = wid * rows_per_w + c * _GCH
            pltpu.sync_copy(idx_hbm.at[pl.ds(base, _GCH)], idx_v)
            pltpu.async_copy(y_hbm.at[idx_v], rows_v, sem).wait()
            pltpu.sync_copy(rows_v, out_hbm.at[pl.ds(base, _GCH)])

    return k(y_sorted, pos_cat)


def _pair_add_body(a_ref, b_ref, o_ref):
    o_ref[...] = a_ref[...] + b_ref[...]


def _pair_add(sel):
    return pl.pallas_call(
        _pair_add_body,
        grid=(_T // _BT,),
        in_specs=[
            pl.BlockSpec((_BT, _D), lambda i: (i, 0)),
            pl.BlockSpec((_BT, _D), lambda i: (_T // _BT + i, 0)),
        ],
        out_specs=pl.BlockSpec((_BT, _D), lambda i: (i, 0)),
        out_shape=jax.ShapeDtypeStruct((_T, _D), jnp.float32),
    )(sel, sel)


def kernel(x, gate_w1, gate_b1, gate_w2, gate_b2,
           exp_w1, exp_b1, exp_w2, exp_b2):
    topk_idx, topk_w = _gating(x, gate_w1, gate_b1, gate_w2, gate_b2)

    flat_e = topk_idx.reshape(-1)
    oh = (flat_e[:, None] == jnp.arange(_E)).astype(jnp.int32)
    rank = jnp.take_along_axis(jnp.cumsum(oh, axis=0) - oh,
                               flat_e[:, None], axis=1)[:, 0]
    counts = jnp.sum(oh, axis=0)
    padded = ((counts + _BR - 1) // _BR) * _BR
    poff = jnp.concatenate([jnp.zeros((1,), jnp.int32),
                            jnp.cumsum(padded).astype(jnp.int32)])
    pos = poff[flat_e] + rank
    tok = jnp.repeat(jnp.arange(_T, dtype=jnp.int32), _K)
    tok_fill = jnp.arange(_LP, dtype=jnp.int32) % _T
    tok_sorted = tok_fill.at[pos].set(tok)
    w_sorted = jnp.zeros((_LP,), jnp.float32).at[pos].set(topk_w.reshape(-1))
    bstart = jnp.arange(_NB, dtype=jnp.int32) * _BR
    block_expert = jnp.clip(
        jnp.searchsorted(poff[1:], bstart, side='right'), 0, _E - 1
    ).astype(jnp.int32)

    x_sorted = _dispatch_gather(x, tok_sorted)

    y_sorted = _ffn(x_sorted, w_sorted, block_expert,
                    exp_w1, exp_b1, exp_w2, exp_b2)

    pos2 = pos.reshape(_T, _K)
    pos_cat = jnp.concatenate([pos2[:, 0], pos2[:, 1]])
    sel = _combine_gather(y_sorted, pos_cat)
    out = _pair_add(sel)
    return (out, topk_idx)


# speedup vs baseline: 1.1338x; 1.1338x over previous
import functools

import jax
import jax.numpy as jnp
from jax import lax
from jax.experimental import pallas as pl
from jax.experimental.pallas import tpu as pltpu
from jax.experimental.pallas import tpu_sc as plsc

_T, _D, _H, _E, _GH, _K = 2048, 1024, 2048, 8, 256, 2
_BT = 256
_BR = 256
_LP = _T * _K + _E * _BR
_NB = _LP // _BR
_NEG = -1e30


def _gating_body(x_ref, gw1_ref, gb1_ref, gw2_ref, gb2_ref, idx_ref, w_ref):
    x = x_ref[...]
    h = jnp.maximum(
        jnp.dot(x, gw1_ref[...], preferred_element_type=jnp.float32)
        + gb1_ref[...], 0.0)
    logits = (jnp.dot(h, gw2_ref[...], preferred_element_type=jnp.float32)
              + gb2_ref[...])
    lane = lax.broadcasted_iota(jnp.int32, logits.shape, 1)
    m1 = jnp.max(logits, axis=-1, keepdims=True)
    i1 = jnp.min(jnp.where(logits == m1, lane, 127), axis=-1, keepdims=True)
    l2 = jnp.where(lane == i1, _NEG, logits)
    m2 = jnp.max(l2, axis=-1, keepdims=True)
    i2 = jnp.min(jnp.where(l2 == m2, lane, 127), axis=-1, keepdims=True)
    e21 = jnp.exp(m2 - m1)
    w1v = 1.0 / (1.0 + e21)
    idx_ref[...] = jnp.concatenate([i1, i2], axis=1)
    w_ref[...] = jnp.concatenate([w1v, 1.0 - w1v], axis=1)


def _gating(x, gw1, gb1, gw2, gb2):
    gw2p = jnp.pad(gw2, ((0, 0), (0, 128 - _E)))
    gb2p = jnp.pad(gb2, (0, 128 - _E), constant_values=_NEG).reshape(1, 128)
    gb1r = gb1.reshape(1, _GH)
    grid = (_T // _BT,)
    return pl.pallas_call(
        _gating_body,
        grid=grid,
        in_specs=[
            pl.BlockSpec((_BT, _D), lambda i: (i, 0)),
            pl.BlockSpec((_D, _GH), lambda i: (0, 0)),
            pl.BlockSpec((1, _GH), lambda i: (0, 0)),
            pl.BlockSpec((_GH, 128), lambda i: (0, 0)),
            pl.BlockSpec((1, 128), lambda i: (0, 0)),
        ],
        out_specs=[
            pl.BlockSpec((_BT, _K), lambda i: (i, 0)),
            pl.BlockSpec((_BT, _K), lambda i: (i, 0)),
        ],
        out_shape=[
            jax.ShapeDtypeStruct((_T, _K), jnp.int32),
            jax.ShapeDtypeStruct((_T, _K), jnp.float32),
        ],
    )(x, gw1, gb1r, gw2p, gb2p)


def _ffn_body(be_ref, xs_ref, w1_ref, b1_ref, w2_ref, b2_ref, ws_ref, y_ref):
    x = xs_ref[...]
    a = jnp.tanh(
        jnp.dot(x, w1_ref[0], preferred_element_type=jnp.float32)
        + b1_ref[0])
    y = (jnp.dot(a, w2_ref[0], preferred_element_type=jnp.float32)
         + b2_ref[0])
    y_ref[...] = y * ws_ref[...]


def _ffn(x_sorted, w_sorted, block_expert, exp_w1, exp_b1, exp_w2, exp_b2):
    grid_spec = pltpu.PrefetchScalarGridSpec(
        num_scalar_prefetch=1,
        grid=(_NB,),
        in_specs=[
            pl.BlockSpec((_BR, _D), lambda b, be: (b, 0)),
            pl.BlockSpec((1, _D, _H), lambda b, be: (be[b], 0, 0)),
            pl.BlockSpec((1, 1, _H), lambda b, be: (be[b], 0, 0)),
            pl.BlockSpec((1, _H, _D), lambda b, be: (be[b], 0, 0)),
            pl.BlockSpec((1, 1, _D), lambda b, be: (be[b], 0, 0)),
            pl.BlockSpec((_BR, 1), lambda b, be: (b, 0)),
        ],
        out_specs=pl.BlockSpec((_BR, _D), lambda b, be: (b, 0)),
    )
    return pl.pallas_call(
        _ffn_body,
        grid_spec=grid_spec,
        out_shape=jax.ShapeDtypeStruct((_LP, _D), jnp.float32),
    )(block_expert, x_sorted, exp_w1, exp_b1.reshape(_E, 1, _H),
      exp_w2, exp_b2.reshape(_E, 1, _D), w_sorted.reshape(_LP, 1))


_NW = 32
_GCH = 64
_CCH = 32


def _sc_wid():
    return lax.axis_index("s") * 2 + lax.axis_index("c")


def _dispatch_gather(x, tok_sorted):
    rows_per_w = _LP // _NW

    @functools.partial(
        pl.kernel,
        out_type=jax.ShapeDtypeStruct((_LP, _D), jnp.float32),
        mesh=plsc.VectorSubcoreMesh(core_axis_name="c", subcore_axis_name="s"),
        scratch_types=[
            pltpu.VMEM((_GCH,), jnp.int32),
            pltpu.VMEM((_GCH, _D), jnp.float32),
            pltpu.SemaphoreType.DMA,
        ],
    )
    def k(x_hbm, idx_hbm, out_hbm, idx_v, rows_v, sem):
        wid = _sc_wid()
        for c in range(rows_per_w // _GCH):
            base = wid * rows_per_w + c * _GCH
            pltpu.sync_copy(idx_hbm.at[pl.ds(base, _GCH)], idx_v)
            pltpu.async_copy(x_hbm.at[idx_v], rows_v, sem).wait()
            pltpu.sync_copy(rows_v, out_hbm.at[pl.ds(base, _GCH)])

    return k(x, tok_sorted)


def _combine_gather(y_sorted, pos_cat):
    rows_per_w = (_K * _T) // _NW

    @functools.partial(
        pl.kernel,
        out_type=jax.ShapeDtypeStruct((_K * _T, _D), jnp.float32),
        mesh=plsc.VectorSubcoreMesh(core_axis_name="c", subcore_axis_name="s"),
        scratch_types=[
            pltpu.VMEM((_GCH,), jnp.int32),
            pltpu.VMEM((_GCH, _D), jnp.float32),
            pltpu.SemaphoreType.DMA,
        ],
    )
    def k(y_hbm, idx_hbm, out_hbm, idx_v, rows_v, sem):
        wid = _sc_wid()
        for c in range(rows_per_w // _GCH):
            base = wid * rows_per_w + c * _GCH
            pltpu.sync_copy(idx_hbm.at[pl.ds(base, _GCH)], idx_v)
            pltpu.async_copy(y_hbm.at[idx_v], rows_v, sem).wait()
            pltpu.sync_copy(rows_v, out_hbm.at[pl.ds(base, _GCH)])

    return k(y_sorted, pos_cat)


def _pair_add_body(a_ref, b_ref, o_ref):
    o_ref[...] = a_ref[...] + b_ref[...]


def _pair_add(sel):
    return pl.pallas_call(
        _pair_add_body,
        grid=(_T // _BT,),
        in_specs=[
            pl.BlockSpec((_BT, _D), lambda i: (i, 0)),
            pl.BlockSpec((_BT, _D), lambda i: (_T // _BT + i, 0)),
        ],
        out_specs=pl.BlockSpec((_BT, _D), lambda i: (i, 0)),
        out_shape=jax.ShapeDtypeStruct((_T, _D), jnp.float32),
    )(sel, sel)


def kernel(x, gate_w1, gate_b1, gate_w2, gate_b2,
           exp_w1, exp_b1, exp_w2, exp_b2):
    topk_idx, topk_w = _gating(x, gate_w1, gate_b1, gate_w2, gate_b2)

    flat_e = topk_idx.reshape(-1)
    oh = (flat_e[:, None] == jnp.arange(_E)).astype(jnp.int32)
    rank = jnp.take_along_axis(jnp.cumsum(oh, axis=0) - oh,
                               flat_e[:, None], axis=1)[:, 0]
    counts = jnp.sum(oh, axis=0)
    padded = ((counts + _BR - 1) // _BR) * _BR
    poff = jnp.concatenate([jnp.zeros((1,), jnp.int32),
                            jnp.cumsum(padded).astype(jnp.int32)])
    pos = poff[flat_e] + rank
    tok = jnp.repeat(jnp.arange(_T, dtype=jnp.int32), _K)
    tok_fill = jnp.arange(_LP, dtype=jnp.int32) % _T
    tok_sorted = tok_fill.at[pos].set(tok)
    w_sorted = jnp.zeros((_LP,), jnp.float32).at[pos].set(topk_w.reshape(-1))
    bstart = jnp.arange(_NB, dtype=jnp.int32) * _BR
    block_expert = jnp.clip(
        jnp.searchsorted(poff[1:], bstart, side='right'), 0, _E - 1
    ).astype(jnp.int32)

    x_sorted = _dispatch_gather(x, tok_sorted)

    y_sorted = _ffn(x_sorted, w_sorted, block_expert,
                    exp_w1, exp_b1, exp_w2, exp_b2)

    pos2 = pos.reshape(_T, _K)
    pos_cat = jnp.concatenate([pos2[:, 0], pos2[:, 1]])
    sel = _combine_gather(y_sorted, pos_cat)
    out = _pair_add(sel)
    return (out, topk_idx)
